# Initial kernel scaffold; baseline (speedup 1.0000x reference)
import jax
import jax.numpy as jnp
from jax.experimental import pallas as pl


def kernel(x, edge_index, edge_attr, W1, b1, W2, b2, We1, be1, We2, be2, We3, be3, We4, be4, Wp1, bp1, Wp2, bp2, Wp3, bp3):
    raise NotImplementedError("write your pallas kernel here")


# speedup vs baseline: 1.8576x; 1.8576x over previous
import jax
import jax.numpy as jnp
from jax.experimental import pallas as pl

N = 10000
E = 320000
D_IN = 128
D_EDGE = 16
H = 128

N_PAD = 10240
E_PAD = 323584
BE = 4096


def _node1_body(cnt_ref, x_ref, w1_ref, dis_ref, hs1_ref):
    cnt = cnt_ref[...]
    dis = jax.lax.rsqrt(cnt + 1.0)
    dis_ref[...] = dis
    hs1_ref[...] = dis * jnp.dot(x_ref[...], w1_ref[...],
                                 preferred_element_type=jnp.float32)


def _node2_body(agg_ref, hs1_ref, dis_ref, b1_ref, w2_ref, hs2_ref):
    agg = agg_ref[0] + agg_ref[1] + hs1_ref[...]
    dis = dis_ref[...]
    h1 = jax.nn.relu(dis * agg + b1_ref[...])
    hs2_ref[...] = dis * jnp.dot(h1, w2_ref[...],
                                 preferred_element_type=jnp.float32)


def _node3_body(agg_ref, hs2_ref, dis_ref, b2_ref, wpa_ref, wpb_ref,
                a_ref, b_ref):
    agg = agg_ref[0] + agg_ref[1] + hs2_ref[...]
    dis = dis_ref[...]
    h2 = jax.nn.relu(dis * agg + b2_ref[...])
    a_ref[...] = jnp.dot(h2, wpa_ref[...], preferred_element_type=jnp.float32)
    b_ref[...] = jnp.dot(h2, wpb_ref[...], preferred_element_type=jnp.float32)


def _edge_body(ea_ref, hr_ref, hc_ref,
               we1_ref, be1_ref, we2_ref, be2_ref, we3_ref, be3_ref,
               we4_ref, be4_ref, wpc_ref, bp1_ref, wp2_ref, bp2_ref,
               wp3_ref, bp3_ref, out_ref):
    f32 = jnp.float32
    e = jax.nn.relu(jnp.dot(ea_ref[...], we1_ref[...],
                            preferred_element_type=f32) + be1_ref[...])
    e = jax.nn.relu(jnp.dot(e, we2_ref[...],
                            preferred_element_type=f32) + be2_ref[...])
    e = jax.nn.relu(jnp.dot(e, we3_ref[...],
                            preferred_element_type=f32) + be3_ref[...])
    e = jax.nn.relu(jnp.dot(e, we4_ref[...],
                            preferred_element_type=f32) + be4_ref[...])
    p = jax.nn.relu(hr_ref[...] + hc_ref[...]
                    + jnp.dot(e, wpc_ref[...], preferred_element_type=f32)
                    + bp1_ref[...])
    p = jax.nn.relu(jnp.dot(p, wp2_ref[...],
                            preferred_element_type=f32) + bp2_ref[...])
    out_ref[...] = (jnp.sum(p * wp3_ref[...], axis=1, keepdims=True)
                    + bp3_ref[...])


def _full(shape):
    nd = len(shape)
    return pl.BlockSpec(shape, lambda *_: (0,) * nd)


def _tc_node1(cnt, x, W1):
    return pl.pallas_call(
        _node1_body,
        grid=(1,),
        in_specs=[_full((N_PAD, 1)), _full((N_PAD, D_IN)), _full((D_IN, H))],
        out_specs=[_full((N_PAD, 1)), _full((N_PAD, H))],
        out_shape=[jax.ShapeDtypeStruct((N_PAD, 1), jnp.float32),
                   jax.ShapeDtypeStruct((N_PAD, H), jnp.float32)],
    )(cnt, x, W1)


def _tc_node2(agg, hs1, dis, b1, W2):
    return pl.pallas_call(
        _node2_body,
        grid=(1,),
        in_specs=[_full((2, N_PAD, H)), _full((N_PAD, H)), _full((N_PAD, 1)),
                  _full((1, H)), _full((H, H))],
        out_specs=[_full((N_PAD, H))],
        out_shape=[jax.ShapeDtypeStruct((N_PAD, H), jnp.float32)],
    )(agg, hs1, dis, b1, W2)[0]


def _tc_node3(agg, hs2, dis, b2, Wpa, Wpb):
    return pl.pallas_call(
        _node3_body,
        grid=(1,),
        in_specs=[_full((2, N_PAD, H)), _full((N_PAD, H)), _full((N_PAD, 1)),
                  _full((1, H)), _full((H, H)), _full((H, H))],
        out_specs=[_full((N_PAD, H)), _full((N_PAD, H))],
        out_shape=[jax.ShapeDtypeStruct((N_PAD, H), jnp.float32),
                   jax.ShapeDtypeStruct((N_PAD, H), jnp.float32)],
    )(agg, hs2, dis, b2, Wpa, Wpb)


def _tc_edge(ea, hr, hc, We1, be1, We2, be2, We3, be3, We4, be4,
             Wpc, bp1, Wp2, bp2, wp3_row, bp3):
    nblk = E_PAD // BE
    eb = pl.BlockSpec((BE, D_EDGE), lambda i: (i, 0))
    hb = pl.BlockSpec((BE, H), lambda i: (i, 0))
    return pl.pallas_call(
        _edge_body,
        grid=(nblk,),
        in_specs=[eb, hb, hb,
                  _full((D_EDGE, H)), _full((1, H)),
                  _full((H, H)), _full((1, H)),
                  _full((H, H)), _full((1, H)),
                  _full((H, H)), _full((1, H)),
                  _full((H, H)), _full((1, H)),
                  _full((H, H)), _full((1, H)),
                  _full((1, H)), _full((1, 1))],
        out_specs=[pl.BlockSpec((BE, 1), lambda i: (i, 0))],
        out_shape=[jax.ShapeDtypeStruct((E_PAD, 1), jnp.float32)],
    )(ea, hr, hc, We1, be1, We2, be2, We3, be3, We4, be4,
      Wpc, bp1, Wp2, bp2, wp3_row, bp3)[0]


def _sc_degree(dst_pad):
    cnt = jnp.zeros((N_PAD,), jnp.float32).at[dst_pad].add(
        1.0, mode="drop", indices_are_sorted=False)
    return cnt


def _sc_aggregate(hs, src_pad, dst_pad):
    half = E_PAD // 2
    agg0 = jnp.zeros((N_PAD, H), jnp.float32).at[dst_pad[:half]].add(
        hs[src_pad[:half]], mode="drop")
    agg1 = jnp.zeros((N_PAD, H), jnp.float32).at[dst_pad[half:]].add(
        hs[src_pad[half:]], mode="drop")
    return jnp.stack([agg0, agg1])


def _sc_gather(table, idx_pad):
    return table[idx_pad]


def kernel(x, edge_index, edge_attr, W1, b1, W2, b2, We1, be1, We2, be2,
           We3, be3, We4, be4, Wp1, bp1, Wp2, bp2, Wp3, bp3):
    row = edge_index[0]
    col = edge_index[1]
    pad_idx = jnp.full((E_PAD - E,), N, jnp.int32)
    src_pad = jnp.concatenate([row, pad_idx])
    dst_pad = jnp.concatenate([col, pad_idx])

    x_pad = jnp.zeros((N_PAD, D_IN), jnp.float32).at[:N].set(x)
    ea_pad = jnp.zeros((E_PAD, D_EDGE), jnp.float32).at[:E].set(edge_attr)

    cnt = _sc_degree(dst_pad)

    dis, hs1 = _tc_node1(cnt[:, None], x_pad, W1)
    agg1 = _sc_aggregate(hs1, src_pad, dst_pad)
    hs2 = _tc_node2(agg1, hs1, dis, b1[None, :], W2)
    agg2 = _sc_aggregate(hs2, src_pad, dst_pad)
    A, B = _tc_node3(agg2, hs2, dis, b2[None, :], Wp1[:H], Wp1[H:2 * H])

    hr = _sc_gather(A, src_pad)
    hc = _sc_gather(B, dst_pad)

    out = _tc_edge(ea_pad, hr, hc, We1, be1[None, :], We2, be2[None, :],
                   We3, be3[None, :], We4, be4[None, :],
                   Wp1[2 * H:], bp1[None, :], Wp2, bp2[None, :],
                   Wp3.T, bp3[None, :])
    return out[:E]


# speedup vs baseline: 5.7968x; 3.1206x over previous
import functools

import jax
import jax.numpy as jnp
from jax import lax
from jax.experimental import pallas as pl
from jax.experimental.pallas import tpu as pltpu
from jax.experimental.pallas import tpu_sc as plsc

N = 10000
E = 320000
D_IN = 128
D_EDGE = 16
H = 128

N_PAD = 10240
E_PAD = 323584
BE = 4096


def _node1_body(cnt_ref, x_ref, w1_ref, dis_ref, hs1_ref):
    cnt = cnt_ref[0] + cnt_ref[1]
    dis = jax.lax.rsqrt(cnt + 1.0)
    dis_ref[...] = dis
    hs1_ref[...] = dis * jnp.dot(x_ref[...], w1_ref[...],
                                 preferred_element_type=jnp.float32)


def _node2_body(agg_ref, hs1_ref, dis_ref, b1_ref, w2_ref, hs2_ref):
    agg = agg_ref[0] + agg_ref[1] + hs1_ref[...]
    dis = dis_ref[...]
    h1 = jax.nn.relu(dis * agg + b1_ref[...])
    hs2_ref[...] = dis * jnp.dot(h1, w2_ref[...],
                                 preferred_element_type=jnp.float32)


def _node3_body(agg_ref, hs2_ref, dis_ref, b2_ref, wpa_ref, wpb_ref,
                a_ref, b_ref):
    agg = agg_ref[0] + agg_ref[1] + hs2_ref[...]
    dis = dis_ref[...]
    h2 = jax.nn.relu(dis * agg + b2_ref[...])
    a_ref[...] = jnp.dot(h2, wpa_ref[...], preferred_element_type=jnp.float32)
    b_ref[...] = jnp.dot(h2, wpb_ref[...], preferred_element_type=jnp.float32)


def _edge_body(ea_ref, hr_ref, hc_ref,
               we1_ref, be1_ref, we2_ref, be2_ref, we3_ref, be3_ref,
               we4_ref, be4_ref, wpc_ref, bp1_ref, wp2_ref, bp2_ref,
               wp3_ref, bp3_ref, out_ref):
    f32 = jnp.float32
    e = jax.nn.relu(jnp.dot(ea_ref[...], we1_ref[...],
                            preferred_element_type=f32) + be1_ref[...])
    e = jax.nn.relu(jnp.dot(e, we2_ref[...],
                            preferred_element_type=f32) + be2_ref[...])
    e = jax.nn.relu(jnp.dot(e, we3_ref[...],
                            preferred_element_type=f32) + be3_ref[...])
    e = jax.nn.relu(jnp.dot(e, we4_ref[...],
                            preferred_element_type=f32) + be4_ref[...])
    p = jax.nn.relu(hr_ref[...] + hc_ref[...]
                    + jnp.dot(e, wpc_ref[...], preferred_element_type=f32)
                    + bp1_ref[...])
    p = jax.nn.relu(jnp.dot(p, wp2_ref[...],
                            preferred_element_type=f32) + bp2_ref[...])
    out_ref[...] = (jnp.sum(p * wp3_ref[...], axis=1, keepdims=True)
                    + bp3_ref[...])


def _full(shape):
    nd = len(shape)
    return pl.BlockSpec(shape, lambda *_: (0,) * nd)


def _tc_node1(cnt, x, W1):
    return pl.pallas_call(
        _node1_body,
        grid=(1,),
        in_specs=[_full((2, N_PAD, 1)), _full((N_PAD, D_IN)), _full((D_IN, H))],
        out_specs=[_full((N_PAD, 1)), _full((N_PAD, H))],
        out_shape=[jax.ShapeDtypeStruct((N_PAD, 1), jnp.float32),
                   jax.ShapeDtypeStruct((N_PAD, H), jnp.float32)],
    )(cnt, x, W1)


def _tc_node2(agg, hs1, dis, b1, W2):
    return pl.pallas_call(
        _node2_body,
        grid=(1,),
        in_specs=[_full((2, N_PAD, H)), _full((N_PAD, H)), _full((N_PAD, 1)),
                  _full((1, H)), _full((H, H))],
        out_specs=[_full((N_PAD, H))],
        out_shape=[jax.ShapeDtypeStruct((N_PAD, H), jnp.float32)],
    )(agg, hs1, dis, b1, W2)[0]


def _tc_node3(agg, hs2, dis, b2, Wpa, Wpb):
    return pl.pallas_call(
        _node3_body,
        grid=(1,),
        in_specs=[_full((2, N_PAD, H)), _full((N_PAD, H)), _full((N_PAD, 1)),
                  _full((1, H)), _full((H, H)), _full((H, H))],
        out_specs=[_full((N_PAD, H)), _full((N_PAD, H))],
        out_shape=[jax.ShapeDtypeStruct((N_PAD, H), jnp.float32),
                   jax.ShapeDtypeStruct((N_PAD, H), jnp.float32)],
    )(agg, hs2, dis, b2, Wpa, Wpb)


def _tc_edge(ea, hr, hc, We1, be1, We2, be2, We3, be3, We4, be4,
             Wpc, bp1, Wp2, bp2, wp3_row, bp3):
    nblk = E_PAD // BE
    eb = pl.BlockSpec((BE, D_EDGE), lambda i: (i, 0))
    hb = pl.BlockSpec((BE, H), lambda i: (i, 0))
    return pl.pallas_call(
        _edge_body,
        grid=(nblk,),
        in_specs=[eb, hb, hb,
                  _full((D_EDGE, H)), _full((1, H)),
                  _full((H, H)), _full((1, H)),
                  _full((H, H)), _full((1, H)),
                  _full((H, H)), _full((1, H)),
                  _full((H, H)), _full((1, H)),
                  _full((H, H)), _full((1, H)),
                  _full((1, H)), _full((1, 1))],
        out_specs=[pl.BlockSpec((BE, 1), lambda i: (i, 0))],
        out_shape=[jax.ShapeDtypeStruct((E_PAD, 1), jnp.float32)],
    )(ea, hr, hc, We1, be1, We2, be2, We3, be3, We4, be4,
      Wpc, bp1, Wp2, bp2, wp3_row, bp3)[0]


_CH = 128
_NW = 32
_NCHUNK = E_PAD // (_NW * _CH)
_RPT = N_PAD // 16


def _sc_mesh():
    return plsc.VectorSubcoreMesh(core_axis_name="c", subcore_axis_name="s")


def _sc_degree(dst_pad, ones_blk, zeros_vec):
    @functools.partial(
        pl.kernel, mesh=_sc_mesh(),
        out_type=jax.ShapeDtypeStruct((2, N_PAD), jnp.float32),
        scratch_types=[pltpu.VMEM((_CH,), jnp.int32),
                       pltpu.VMEM((_CH,), jnp.float32),
                       pltpu.VMEM_SHARED((N_PAD,), jnp.float32),
                       pltpu.SemaphoreType.DMA])
    def k(dst_hbm, ones_hbm, z_hbm, out_hbm, idx_d, ones_v, acc, sem):
        c = lax.axis_index("c")
        s = lax.axis_index("s")
        pltpu.sync_copy(z_hbm, acc.at[pl.ds(s * _RPT, _RPT)])
        pltpu.sync_copy(ones_hbm, ones_v)
        plsc.subcore_barrier()

        @pl.loop(0, _NCHUNK)
        def _(i):
            base = (c * 16 + s) * (_NCHUNK * _CH) + i * _CH
            pltpu.sync_copy(dst_hbm.at[pl.ds(base, _CH)], idx_d)
            pltpu.sync_copy(ones_v, acc.at[idx_d], add=True)

        plsc.subcore_barrier()
        pltpu.sync_copy(acc.at[pl.ds(s * _RPT, _RPT)],
                        out_hbm.at[c, pl.ds(s * _RPT, _RPT)])

    return k(dst_pad, ones_blk, zeros_vec)


def _sc_aggregate(hs, src_pad, dst_pad, zeros_blk):
    @functools.partial(
        pl.kernel, mesh=_sc_mesh(),
        out_type=jax.ShapeDtypeStruct((2, N_PAD, H), jnp.float32),
        scratch_types=[pltpu.VMEM((_CH,), jnp.int32),
                       pltpu.VMEM((_CH,), jnp.int32),
                       pltpu.VMEM((_CH, H), jnp.float32),
                       pltpu.VMEM_SHARED((N_PAD, H), jnp.float32),
                       pltpu.SemaphoreType.DMA])
    def k(hs_hbm, src_hbm, dst_hbm, z_hbm, out_hbm, idx_s, idx_d, rows, acc,
          sem):
        c = lax.axis_index("c")
        s = lax.axis_index("s")
        pltpu.sync_copy(z_hbm, acc.at[pl.ds(s * _RPT, _RPT)])
        plsc.subcore_barrier()

        @pl.loop(0, _NCHUNK)
        def _(i):
            base = (c * 16 + s) * (_NCHUNK * _CH) + i * _CH
            pltpu.sync_copy(src_hbm.at[pl.ds(base, _CH)], idx_s)
            pltpu.async_copy(hs_hbm.at[idx_s], rows, sem).wait()
            pltpu.sync_copy(dst_hbm.at[pl.ds(base, _CH)], idx_d)
            pltpu.sync_copy(rows, acc.at[idx_d], add=True)

        plsc.subcore_barrier()
        pltpu.sync_copy(acc.at[pl.ds(s * _RPT, _RPT)],
                        out_hbm.at[c, pl.ds(s * _RPT, _RPT)])

    return k(hs, src_pad, dst_pad, zeros_blk)


def _sc_gather_pair(a, b, src_pad, dst_pad):
    @functools.partial(
        pl.kernel, mesh=_sc_mesh(),
        out_type=[jax.ShapeDtypeStruct((E_PAD, H), jnp.float32),
                  jax.ShapeDtypeStruct((E_PAD, H), jnp.float32)],
        scratch_types=[pltpu.VMEM((_CH,), jnp.int32),
                       pltpu.VMEM((_CH,), jnp.int32),
                       pltpu.VMEM((_CH, H), jnp.float32),
                       pltpu.VMEM((_CH, H), jnp.float32),
                       pltpu.SemaphoreType.DMA,
                       pltpu.SemaphoreType.DMA])
    def k(a_hbm, b_hbm, src_hbm, dst_hbm, hr_hbm, hc_hbm,
          idx1, idx2, buf1, buf2, sem1, sem2):
        c = lax.axis_index("c")
        s = lax.axis_index("s")
        wid = c * 16 + s

        @pl.loop(0, _NCHUNK)
        def _(i):
            base = (wid * _NCHUNK + i) * _CH
            pltpu.sync_copy(src_hbm.at[pl.ds(base, _CH)], idx1)
            pltpu.sync_copy(dst_hbm.at[pl.ds(base, _CH)], idx2)
            cp1 = pltpu.async_copy(a_hbm.at[idx1], buf1, sem1)
            cp2 = pltpu.async_copy(b_hbm.at[idx2], buf2, sem2)
            cp1.wait()
            cp2.wait()
            pltpu.sync_copy(buf1, hr_hbm.at[pl.ds(base, _CH)])
            pltpu.sync_copy(buf2, hc_hbm.at[pl.ds(base, _CH)])

    return k(a, b, src_pad, dst_pad)


def kernel(x, edge_index, edge_attr, W1, b1, W2, b2, We1, be1, We2, be2,
           We3, be3, We4, be4, Wp1, bp1, Wp2, bp2, Wp3, bp3):
    row = edge_index[0]
    col = edge_index[1]
    pad_idx = jnp.full((E_PAD - E,), N, jnp.int32)
    src_pad = jnp.concatenate([row, pad_idx])
    dst_pad = jnp.concatenate([col, pad_idx])

    x_pad = jnp.zeros((N_PAD, D_IN), jnp.float32).at[:N].set(x)
    ea_pad = jnp.zeros((E_PAD, D_EDGE), jnp.float32).at[:E].set(edge_attr)

    ones_blk = jnp.ones((_CH,), jnp.float32)
    zeros_vec = jnp.zeros((_RPT,), jnp.float32)
    zeros_blk = jnp.zeros((_RPT, H), jnp.float32)

    cnt = _sc_degree(dst_pad, ones_blk, zeros_vec)

    dis, hs1 = _tc_node1(cnt[:, :, None], x_pad, W1)
    agg1 = _sc_aggregate(hs1, src_pad, dst_pad, zeros_blk)
    hs2 = _tc_node2(agg1, hs1, dis, b1[None, :], W2)
    agg2 = _sc_aggregate(hs2, src_pad, dst_pad, zeros_blk)
    A, B = _tc_node3(agg2, hs2, dis, b2[None, :], Wp1[:H], Wp1[H:2 * H])

    hr, hc = _sc_gather_pair(A, B, src_pad, dst_pad)

    out = _tc_edge(ea_pad, hr, hc, We1, be1[None, :], We2, be2[None, :],
                   We3, be3[None, :], We4, be4[None, :],
                   Wp1[2 * H:], bp1[None, :], Wp2, bp2[None, :],
                   Wp3.T, bp3[None, :])
    return out[:E]


# speedup vs baseline: 6.2373x; 1.0760x over previous
import functools

import jax
import jax.numpy as jnp
from jax import lax
from jax.experimental import pallas as pl
from jax.experimental.pallas import tpu as pltpu
from jax.experimental.pallas import tpu_sc as plsc

N = 10000
E = 320000
D_IN = 128
D_EDGE = 16
H = 128

N_PAD = 10240
E_PAD = 323584
BE = 4096


def _node1_body(cnt_ref, x_ref, w1_ref, dis_ref, hs1_ref):
    cnt = cnt_ref[0] + cnt_ref[1]
    dis = jax.lax.rsqrt(cnt + 1.0)
    dis_ref[...] = dis
    hs1_ref[...] = dis * jnp.dot(x_ref[...], w1_ref[...],
                                 preferred_element_type=jnp.float32)


def _node2_body(agg_ref, hs1_ref, dis_ref, b1_ref, w2_ref, hs2_ref):
    agg = agg_ref[0] + agg_ref[1] + hs1_ref[...]
    dis = dis_ref[...]
    h1 = jax.nn.relu(dis * agg + b1_ref[...])
    hs2_ref[...] = dis * jnp.dot(h1, w2_ref[...],
                                 preferred_element_type=jnp.float32)


def _node3_body(agg_ref, hs2_ref, dis_ref, b2_ref, wpa_ref, wpb_ref,
                a_ref, b_ref):
    agg = agg_ref[0] + agg_ref[1] + hs2_ref[...]
    dis = dis_ref[...]
    h2 = jax.nn.relu(dis * agg + b2_ref[...])
    a_ref[...] = jnp.dot(h2, wpa_ref[...], preferred_element_type=jnp.float32)
    b_ref[...] = jnp.dot(h2, wpb_ref[...], preferred_element_type=jnp.float32)


def _edge_body(ea_ref, hr_ref, hc_ref,
               we1_ref, be1_ref, we2_ref, be2_ref, we3_ref, be3_ref,
               we4_ref, be4_ref, wpc_ref, bp1_ref, wp2_ref, bp2_ref,
               wp3_ref, bp3_ref, out_ref):
    f32 = jnp.float32
    e = jax.nn.relu(jnp.dot(ea_ref[...], we1_ref[...],
                            preferred_element_type=f32) + be1_ref[...])
    e = jax.nn.relu(jnp.dot(e, we2_ref[...],
                            preferred_element_type=f32) + be2_ref[...])
    e = jax.nn.relu(jnp.dot(e, we3_ref[...],
                            preferred_element_type=f32) + be3_ref[...])
    e = jax.nn.relu(jnp.dot(e, we4_ref[...],
                            preferred_element_type=f32) + be4_ref[...])
    p = jax.nn.relu(hr_ref[...] + hc_ref[...]
                    + jnp.dot(e, wpc_ref[...], preferred_element_type=f32)
                    + bp1_ref[...])
    p = jax.nn.relu(jnp.dot(p, wp2_ref[...],
                            preferred_element_type=f32) + bp2_ref[...])
    out_ref[...] = (jnp.sum(p * wp3_ref[...], axis=1, keepdims=True)
                    + bp3_ref[...])


def _full(shape):
    nd = len(shape)
    return pl.BlockSpec(shape, lambda *_: (0,) * nd)


def _tc_node1(cnt, x, W1):
    return pl.pallas_call(
        _node1_body,
        grid=(1,),
        in_specs=[_full((2, N_PAD, 1)), _full((N_PAD, D_IN)), _full((D_IN, H))],
        out_specs=[_full((N_PAD, 1)), _full((N_PAD, H))],
        out_shape=[jax.ShapeDtypeStruct((N_PAD, 1), jnp.float32),
                   jax.ShapeDtypeStruct((N_PAD, H), jnp.float32)],
    )(cnt, x, W1)


def _tc_node2(agg, hs1, dis, b1, W2):
    return pl.pallas_call(
        _node2_body,
        grid=(1,),
        in_specs=[_full((2, N_PAD, H)), _full((N_PAD, H)), _full((N_PAD, 1)),
                  _full((1, H)), _full((H, H))],
        out_specs=[_full((N_PAD, H))],
        out_shape=[jax.ShapeDtypeStruct((N_PAD, H), jnp.float32)],
    )(agg, hs1, dis, b1, W2)[0]


def _tc_node3(agg, hs2, dis, b2, Wpa, Wpb):
    return pl.pallas_call(
        _node3_body,
        grid=(1,),
        in_specs=[_full((2, N_PAD, H)), _full((N_PAD, H)), _full((N_PAD, 1)),
                  _full((1, H)), _full((H, H)), _full((H, H))],
        out_specs=[_full((N_PAD, H)), _full((N_PAD, H))],
        out_shape=[jax.ShapeDtypeStruct((N_PAD, H), jnp.float32),
                   jax.ShapeDtypeStruct((N_PAD, H), jnp.float32)],
    )(agg, hs2, dis, b2, Wpa, Wpb)


def _tc_edge(ea, hr, hc, We1, be1, We2, be2, We3, be3, We4, be4,
             Wpc, bp1, Wp2, bp2, wp3_row, bp3):
    nblk = E_PAD // BE
    eb = pl.BlockSpec((BE, D_EDGE), lambda i: (i, 0))
    hb = pl.BlockSpec((BE, H), lambda i: (i, 0))
    return pl.pallas_call(
        _edge_body,
        grid=(nblk,),
        in_specs=[eb, hb, hb,
                  _full((D_EDGE, H)), _full((1, H)),
                  _full((H, H)), _full((1, H)),
                  _full((H, H)), _full((1, H)),
                  _full((H, H)), _full((1, H)),
                  _full((H, H)), _full((1, H)),
                  _full((H, H)), _full((1, H)),
                  _full((1, H)), _full((1, 1))],
        out_specs=[pl.BlockSpec((BE, 1), lambda i: (i, 0))],
        out_shape=[jax.ShapeDtypeStruct((E_PAD, 1), jnp.float32)],
    )(ea, hr, hc, We1, be1, We2, be2, We3, be3, We4, be4,
      Wpc, bp1, Wp2, bp2, wp3_row, bp3)[0]


_CH = 128
_NW = 32
_NCHUNK = E_PAD // (_NW * _CH)
_RPT = N_PAD // 16


def _sc_mesh():
    return plsc.VectorSubcoreMesh(core_axis_name="c", subcore_axis_name="s")


def _sc_degree(dst_pad, ones_blk, zeros_vec):
    @functools.partial(
        pl.kernel, mesh=_sc_mesh(),
        out_type=jax.ShapeDtypeStruct((2, N_PAD), jnp.float32),
        scratch_types=[pltpu.VMEM((_CH,), jnp.int32),
                       pltpu.VMEM((_CH,), jnp.float32),
                       pltpu.VMEM_SHARED((N_PAD,), jnp.float32),
                       pltpu.SemaphoreType.DMA])
    def k(dst_hbm, ones_hbm, z_hbm, out_hbm, idx_d, ones_v, acc, sem):
        c = lax.axis_index("c")
        s = lax.axis_index("s")
        pltpu.sync_copy(z_hbm, acc.at[pl.ds(s * _RPT, _RPT)])
        pltpu.sync_copy(ones_hbm, ones_v)
        plsc.subcore_barrier()

        @pl.loop(0, _NCHUNK)
        def _(i):
            base = (c * 16 + s) * (_NCHUNK * _CH) + i * _CH
            pltpu.sync_copy(dst_hbm.at[pl.ds(base, _CH)], idx_d)
            pltpu.sync_copy(ones_v, acc.at[idx_d], add=True)

        plsc.subcore_barrier()
        pltpu.sync_copy(acc.at[pl.ds(s * _RPT, _RPT)],
                        out_hbm.at[c, pl.ds(s * _RPT, _RPT)])

    return k(dst_pad, ones_blk, zeros_vec)


def _sc_aggregate(hs, src_pad, dst_pad, zeros_blk):
    @functools.partial(
        pl.kernel, mesh=_sc_mesh(),
        out_type=jax.ShapeDtypeStruct((2, N_PAD, H), jnp.float32),
        scratch_types=[pltpu.VMEM((2, _CH), jnp.int32),
                       pltpu.VMEM((2, _CH), jnp.int32),
                       pltpu.VMEM((2, _CH, H), jnp.float32),
                       pltpu.VMEM_SHARED((N_PAD, H), jnp.float32),
                       pltpu.SemaphoreType.DMA,
                       pltpu.SemaphoreType.DMA])
    def k(hs_hbm, src_hbm, dst_hbm, z_hbm, out_hbm, idx_s, idx_d, rows, acc,
          sem0, sem1):
        c = lax.axis_index("c")
        s = lax.axis_index("s")
        base0 = (c * 16 + s) * (_NCHUNK * _CH)
        sems = [sem0, sem1]
        pltpu.sync_copy(z_hbm, acc.at[pl.ds(s * _RPT, _RPT)])
        plsc.subcore_barrier()

        pltpu.sync_copy(src_hbm.at[pl.ds(base0, _CH)], idx_s.at[0])
        pltpu.sync_copy(dst_hbm.at[pl.ds(base0, _CH)], idx_d.at[0])
        pltpu.async_copy(hs_hbm.at[idx_s.at[0]], rows.at[0], sem0)

        @pl.loop(0, _NCHUNK, step=2)
        def _(i):
            for b in range(2):
                nxt = 1 - b

                @pl.when(i + b < _NCHUNK)
                def _():
                    g = i + b
                    pltpu.make_async_copy(hs_hbm.at[idx_s.at[b]],
                                          rows.at[b], sems[b]).wait()

                    @pl.when(g + 1 < _NCHUNK)
                    def _():
                        nb = base0 + (g + 1) * _CH
                        pltpu.sync_copy(src_hbm.at[pl.ds(nb, _CH)],
                                        idx_s.at[nxt])
                        pltpu.sync_copy(dst_hbm.at[pl.ds(nb, _CH)],
                                        idx_d.at[nxt])
                        pltpu.async_copy(hs_hbm.at[idx_s.at[nxt]],
                                         rows.at[nxt], sems[nxt])

                    pltpu.sync_copy(rows.at[b], acc.at[idx_d.at[b]], add=True)

        plsc.subcore_barrier()
        pltpu.sync_copy(acc.at[pl.ds(s * _RPT, _RPT)],
                        out_hbm.at[c, pl.ds(s * _RPT, _RPT)])

    return k(hs, src_pad, dst_pad, zeros_blk)


def _sc_gather_pair(a, b, src_pad, dst_pad):
    @functools.partial(
        pl.kernel, mesh=_sc_mesh(),
        out_type=[jax.ShapeDtypeStruct((E_PAD, H), jnp.float32),
                  jax.ShapeDtypeStruct((E_PAD, H), jnp.float32)],
        scratch_types=[pltpu.VMEM((2, _CH), jnp.int32),
                       pltpu.VMEM((2, _CH), jnp.int32),
                       pltpu.VMEM((2, _CH, H), jnp.float32),
                       pltpu.VMEM((2, _CH, H), jnp.float32),
                       pltpu.SemaphoreType.DMA,
                       pltpu.SemaphoreType.DMA,
                       pltpu.SemaphoreType.DMA,
                       pltpu.SemaphoreType.DMA])
    def k(a_hbm, b_hbm, src_hbm, dst_hbm, hr_hbm, hc_hbm,
          idx1, idx2, buf1, buf2, gsem0, gsem1, wsem0, wsem1):
        c = lax.axis_index("c")
        s = lax.axis_index("s")
        base0 = (c * 16 + s) * (_NCHUNK * _CH)
        gsems = [gsem0, gsem1]
        wsems = [wsem0, wsem1]

        pltpu.sync_copy(src_hbm.at[pl.ds(base0, _CH)], idx1.at[0])
        pltpu.sync_copy(dst_hbm.at[pl.ds(base0, _CH)], idx2.at[0])
        pltpu.async_copy(a_hbm.at[idx1.at[0]], buf1.at[0], gsem0)
        pltpu.async_copy(b_hbm.at[idx2.at[0]], buf2.at[0], gsem0)

        @pl.loop(0, _NCHUNK, step=2)
        def _(i):
            for b in range(2):
                nxt = 1 - b

                @pl.when(i + b < _NCHUNK)
                def _():
                    g = i + b
                    pltpu.make_async_copy(a_hbm.at[idx1.at[b]],
                                          buf1.at[b], gsems[b]).wait()
                    pltpu.make_async_copy(b_hbm.at[idx2.at[b]],
                                          buf2.at[b], gsems[b]).wait()

                    @pl.when(g + 1 < _NCHUNK)
                    def _():
                        nb = base0 + (g + 1) * _CH
                        pltpu.sync_copy(src_hbm.at[pl.ds(nb, _CH)],
                                        idx1.at[nxt])
                        pltpu.sync_copy(dst_hbm.at[pl.ds(nb, _CH)],
                                        idx2.at[nxt])

                        @pl.when(g >= 1)
                        def _():
                            ob = base0 + (g - 1) * _CH
                            pltpu.make_async_copy(
                                buf1.at[nxt], hr_hbm.at[pl.ds(ob, _CH)],
                                wsems[nxt]).wait()
                            pltpu.make_async_copy(
                                buf2.at[nxt], hc_hbm.at[pl.ds(ob, _CH)],
                                wsems[nxt]).wait()

                        pltpu.async_copy(a_hbm.at[idx1.at[nxt]],
                                         buf1.at[nxt], gsems[nxt])
                        pltpu.async_copy(b_hbm.at[idx2.at[nxt]],
                                         buf2.at[nxt], gsems[nxt])

                    ob = base0 + g * _CH
                    pltpu.async_copy(buf1.at[b], hr_hbm.at[pl.ds(ob, _CH)],
                                     wsems[b])
                    pltpu.async_copy(buf2.at[b], hc_hbm.at[pl.ds(ob, _CH)],
                                     wsems[b])

        last = _NCHUNK - 1
        lb = base0 + last * _CH
        pb = base0 + (last - 1) * _CH
        pltpu.make_async_copy(buf1.at[(last - 1) % 2],
                              hr_hbm.at[pl.ds(pb, _CH)],
                              wsems[(last - 1) % 2]).wait()
        pltpu.make_async_copy(buf2.at[(last - 1) % 2],
                              hc_hbm.at[pl.ds(pb, _CH)],
                              wsems[(last - 1) % 2]).wait()
        pltpu.make_async_copy(buf1.at[last % 2], hr_hbm.at[pl.ds(lb, _CH)],
                              wsems[last % 2]).wait()
        pltpu.make_async_copy(buf2.at[last % 2], hc_hbm.at[pl.ds(lb, _CH)],
                              wsems[last % 2]).wait()

    return k(a, b, src_pad, dst_pad)


def kernel(x, edge_index, edge_attr, W1, b1, W2, b2, We1, be1, We2, be2,
           We3, be3, We4, be4, Wp1, bp1, Wp2, bp2, Wp3, bp3):
    row = edge_index[0]
    col = edge_index[1]
    pad_idx = jnp.full((E_PAD - E,), N, jnp.int32)
    src_pad = jnp.concatenate([row, pad_idx])
    dst_pad = jnp.concatenate([col, pad_idx])

    x_pad = jnp.zeros((N_PAD, D_IN), jnp.float32).at[:N].set(x)
    ea_pad = jnp.zeros((E_PAD, D_EDGE), jnp.float32).at[:E].set(edge_attr)

    ones_blk = jnp.ones((_CH,), jnp.float32)
    zeros_vec = jnp.zeros((_RPT,), jnp.float32)
    zeros_blk = jnp.zeros((_RPT, H), jnp.float32)

    cnt = _sc_degree(dst_pad, ones_blk, zeros_vec)

    dis, hs1 = _tc_node1(cnt[:, :, None], x_pad, W1)
    agg1 = _sc_aggregate(hs1, src_pad, dst_pad, zeros_blk)
    hs2 = _tc_node2(agg1, hs1, dis, b1[None, :], W2)
    agg2 = _sc_aggregate(hs2, src_pad, dst_pad, zeros_blk)
    A, B = _tc_node3(agg2, hs2, dis, b2[None, :], Wp1[:H], Wp1[H:2 * H])

    hr, hc = _sc_gather_pair(A, B, src_pad, dst_pad)

    out = _tc_edge(ea_pad, hr, hc, We1, be1[None, :], We2, be2[None, :],
                   We3, be3[None, :], We4, be4[None, :],
                   Wp1[2 * H:], bp1[None, :], Wp2, bp2[None, :],
                   Wp3.T, bp3[None, :])
    return out[:E]


# speedup vs baseline: 6.4125x; 1.0281x over previous
import functools

import jax
import jax.numpy as jnp
from jax import lax
from jax.experimental import pallas as pl
from jax.experimental.pallas import tpu as pltpu
from jax.experimental.pallas import tpu_sc as plsc

N = 10000
E = 320000
D_IN = 128
D_EDGE = 16
H = 128

N_PAD = 10240
E_PAD = 323584
BE = 4096


def _node1_body(cnt_ref, x_ref, w1_ref, dis_ref, hs1_ref):
    cnt = cnt_ref[0] + cnt_ref[1]
    dis = jax.lax.rsqrt(cnt + 1.0)
    dis_ref[...] = dis
    hs1_ref[...] = dis * jnp.dot(x_ref[...], w1_ref[...],
                                 preferred_element_type=jnp.float32)


def _node2_body(agg_ref, hs1_ref, dis_ref, b1_ref, w2_ref, hs2_ref):
    agg = agg_ref[0] + agg_ref[1] + hs1_ref[...]
    dis = dis_ref[...]
    h1 = jax.nn.relu(dis * agg + b1_ref[...])
    hs2_ref[...] = dis * jnp.dot(h1, w2_ref[...],
                                 preferred_element_type=jnp.float32)


def _node3_body(agg_ref, hs2_ref, dis_ref, b2_ref, wpa_ref, wpb_ref,
                a_ref, b_ref):
    agg = agg_ref[0] + agg_ref[1] + hs2_ref[...]
    dis = dis_ref[...]
    h2 = jax.nn.relu(dis * agg + b2_ref[...])
    a_ref[...] = jnp.dot(h2, wpa_ref[...], preferred_element_type=jnp.float32)
    b_ref[...] = jnp.dot(h2, wpb_ref[...], preferred_element_type=jnp.float32)


H2 = H // 2


def _edge_body(ea_ref, hr_ref, hc_ref,
               we1_ref, be1_ref, we2_ref, be2_ref, we3_ref, be3_ref,
               we4_ref, be4_ref, wpc_ref, bp1_ref, wp2_ref, bp2_ref,
               wp3_ref, bp3_ref, out_ref):
    f32 = jnp.float32
    e = jax.nn.relu(jnp.dot(ea_ref[...], we1_ref[...],
                            preferred_element_type=f32) + be1_ref[...])
    e = jax.nn.relu(jnp.dot(e, we2_ref[...],
                            preferred_element_type=f32) + be2_ref[...])
    e = jax.nn.relu(jnp.dot(e, we3_ref[...],
                            preferred_element_type=f32) + be3_ref[...])
    e = jax.nn.relu(jnp.dot(e, we4_ref[...],
                            preferred_element_type=f32) + be4_ref[...])
    p = jax.nn.relu(hr_ref[...] + hc_ref[...]
                    + jnp.dot(e, wpc_ref[...], preferred_element_type=f32)
                    + bp1_ref[...])
    p = jax.nn.relu(jnp.dot(p, wp2_ref[...],
                            preferred_element_type=f32) + bp2_ref[...])
    out_ref[...] = (jnp.sum(p * wp3_ref[...], axis=1, keepdims=True)
                    + bp3_ref[...])


def _full(shape):
    nd = len(shape)
    return pl.BlockSpec(shape, lambda *_: (0,) * nd)


def _tc_node1(cnt, x, W1):
    return pl.pallas_call(
        _node1_body,
        grid=(1,),
        in_specs=[_full((2, N_PAD, 1)), _full((N_PAD, D_IN)), _full((D_IN, H))],
        out_specs=[_full((N_PAD, 1)), _full((N_PAD, H))],
        out_shape=[jax.ShapeDtypeStruct((N_PAD, 1), jnp.float32),
                   jax.ShapeDtypeStruct((N_PAD, H), jnp.float32)],
    )(cnt, x, W1)


def _tc_node2(agg, hs1, dis, b1, W2):
    return pl.pallas_call(
        _node2_body,
        grid=(1,),
        in_specs=[_full((2, N_PAD, H)), _full((N_PAD, H)), _full((N_PAD, 1)),
                  _full((1, H)), _full((H, H))],
        out_specs=[_full((N_PAD, H))],
        out_shape=[jax.ShapeDtypeStruct((N_PAD, H), jnp.float32)],
    )(agg, hs1, dis, b1, W2)[0]


def _tc_node3(agg, hs2, dis, b2, Wpa, Wpb):
    return pl.pallas_call(
        _node3_body,
        grid=(1,),
        in_specs=[_full((2, N_PAD, H)), _full((N_PAD, H)), _full((N_PAD, 1)),
                  _full((1, H)), _full((H, H)), _full((H, H))],
        out_specs=[_full((N_PAD, H)), _full((N_PAD, H))],
        out_shape=[jax.ShapeDtypeStruct((N_PAD, H), jnp.float32),
                   jax.ShapeDtypeStruct((N_PAD, H), jnp.float32)],
    )(agg, hs2, dis, b2, Wpa, Wpb)


def _tc_edge(ea, hr, hc, We1, be1, We2, be2, We3, be3, We4, be4,
             Wpc, bp1, Wp2, bp2, wp3_row, bp3):
    nblk = E_PAD // BE
    eb = pl.BlockSpec((BE, D_EDGE), lambda i: (i, 0))
    hb = pl.BlockSpec((BE, H), lambda i: (i, 0))
    return pl.pallas_call(
        _edge_body,
        grid=(nblk,),
        in_specs=[eb, hb, hb,
                  _full((D_EDGE, H)), _full((1, H)),
                  _full((H, H)), _full((1, H)),
                  _full((H, H)), _full((1, H)),
                  _full((H, H)), _full((1, H)),
                  _full((H, H)), _full((1, H)),
                  _full((H, H)), _full((1, H)),
                  _full((1, H)), _full((1, 1))],
        out_specs=[pl.BlockSpec((BE, 1), lambda i: (i, 0))],
        out_shape=[jax.ShapeDtypeStruct((E_PAD, 1), jnp.float32)],
    )(ea, hr, hc, We1, be1, We2, be2, We3, be3, We4, be4,
      Wpc, bp1, Wp2, bp2, wp3_row, bp3)[0]


_CH = 128
_NW = 32
_RPT = N_PAD // 16

_NCH0 = 92
_NCH1 = 66
_MAXCH = max(_NCH0, _NCH1)


def _my_chunks(c, s):
    n_my = lax.select(c == 0, jnp.int32(_NCH0), jnp.int32(_NCH1))
    chunk0 = lax.select(c == 0, s * _NCH0, 16 * _NCH0 + s * _NCH1)
    return n_my, chunk0 * _CH


def _sc_mesh():
    return plsc.VectorSubcoreMesh(core_axis_name="c", subcore_axis_name="s")


def _sc_degree(dst_pad, ones_blk, zeros_vec):
    @functools.partial(
        pl.kernel, mesh=_sc_mesh(),
        out_type=jax.ShapeDtypeStruct((2, N_PAD), jnp.float32),
        scratch_types=[pltpu.VMEM((_CH,), jnp.int32),
                       pltpu.VMEM((_CH,), jnp.float32),
                       pltpu.VMEM_SHARED((N_PAD,), jnp.float32),
                       pltpu.SemaphoreType.DMA])
    def k(dst_hbm, ones_hbm, z_hbm, out_hbm, idx_d, ones_v, acc, sem):
        c = lax.axis_index("c")
        s = lax.axis_index("s")
        n_my, base0 = _my_chunks(c, s)
        pltpu.sync_copy(z_hbm, acc.at[pl.ds(s * _RPT, _RPT)])
        pltpu.sync_copy(ones_hbm, ones_v)
        plsc.subcore_barrier()

        @pl.loop(0, _MAXCH)
        def _(i):
            @pl.when(i < n_my)
            def _():
                base = base0 + i * _CH
                pltpu.sync_copy(dst_hbm.at[pl.ds(base, _CH)], idx_d)
                pltpu.sync_copy(ones_v, acc.at[idx_d], add=True)

        plsc.subcore_barrier()
        pltpu.sync_copy(acc.at[pl.ds(s * _RPT, _RPT)],
                        out_hbm.at[c, pl.ds(s * _RPT, _RPT)])

    return k(dst_pad, ones_blk, zeros_vec)


def _sc_aggregate(hs, src_pad, dst_pad, zeros_blk):
    @functools.partial(
        pl.kernel, mesh=_sc_mesh(),
        out_type=jax.ShapeDtypeStruct((2, N_PAD, H), jnp.float32),
        scratch_types=[pltpu.VMEM((2, _CH), jnp.int32),
                       pltpu.VMEM((2, _CH), jnp.int32),
                       pltpu.VMEM((2, _CH, H), jnp.float32),
                       pltpu.VMEM_SHARED((N_PAD, H), jnp.float32),
                       pltpu.SemaphoreType.DMA,
                       pltpu.SemaphoreType.DMA])
    def k(hs_hbm, src_hbm, dst_hbm, z_hbm, out_hbm, idx_s, idx_d, rows, acc,
          sem0, sem1):
        c = lax.axis_index("c")
        s = lax.axis_index("s")
        n_my, base0 = _my_chunks(c, s)
        sems = [sem0, sem1]
        pltpu.sync_copy(z_hbm, acc.at[pl.ds(s * _RPT, _RPT)])
        plsc.subcore_barrier()

        pltpu.sync_copy(src_hbm.at[pl.ds(base0, _CH)], idx_s.at[0])
        pltpu.sync_copy(dst_hbm.at[pl.ds(base0, _CH)], idx_d.at[0])
        pltpu.async_copy(hs_hbm.at[idx_s.at[0]], rows.at[0], sem0)

        @pl.loop(0, _MAXCH, step=2)
        def _(i):
            for b in range(2):
                nxt = 1 - b

                @pl.when(i + b < n_my)
                def _():
                    g = i + b
                    pltpu.make_async_copy(hs_hbm.at[idx_s.at[b]],
                                          rows.at[b], sems[b]).wait()

                    @pl.when(g + 1 < n_my)
                    def _():
                        nb = base0 + (g + 1) * _CH
                        pltpu.sync_copy(src_hbm.at[pl.ds(nb, _CH)],
                                        idx_s.at[nxt])
                        pltpu.sync_copy(dst_hbm.at[pl.ds(nb, _CH)],
                                        idx_d.at[nxt])
                        pltpu.async_copy(hs_hbm.at[idx_s.at[nxt]],
                                         rows.at[nxt], sems[nxt])

                    pltpu.sync_copy(rows.at[b], acc.at[idx_d.at[b]], add=True)

        plsc.subcore_barrier()
        pltpu.sync_copy(acc.at[pl.ds(s * _RPT, _RPT)],
                        out_hbm.at[c, pl.ds(s * _RPT, _RPT)])

    return k(hs, src_pad, dst_pad, zeros_blk)


def _sc_gather_pair(a, b, src_pad, dst_pad):
    @functools.partial(
        pl.kernel, mesh=_sc_mesh(),
        out_type=[jax.ShapeDtypeStruct((E_PAD, H), jnp.float32),
                  jax.ShapeDtypeStruct((E_PAD, H), jnp.float32)],
        scratch_types=[pltpu.VMEM((2, _CH), jnp.int32),
                       pltpu.VMEM((2, _CH), jnp.int32),
                       pltpu.VMEM((2, _CH, H), jnp.float32),
                       pltpu.VMEM((2, _CH, H), jnp.float32),
                       pltpu.SemaphoreType.DMA,
                       pltpu.SemaphoreType.DMA,
                       pltpu.SemaphoreType.DMA,
                       pltpu.SemaphoreType.DMA])
    def k(a_hbm, b_hbm, src_hbm, dst_hbm, hr_hbm, hc_hbm,
          idx1, idx2, buf1, buf2, gsem0, gsem1, wsem0, wsem1):
        c = lax.axis_index("c")
        s = lax.axis_index("s")
        n_my, base0 = _my_chunks(c, s)
        gsems = [gsem0, gsem1]
        wsems = [wsem0, wsem1]

        pltpu.sync_copy(src_hbm.at[pl.ds(base0, _CH)], idx1.at[0])
        pltpu.sync_copy(dst_hbm.at[pl.ds(base0, _CH)], idx2.at[0])
        pltpu.async_copy(a_hbm.at[idx1.at[0]], buf1.at[0], gsem0)
        pltpu.async_copy(b_hbm.at[idx2.at[0]], buf2.at[0], gsem0)

        @pl.loop(0, _MAXCH, step=2)
        def _(i):
            for b in range(2):
                nxt = 1 - b

                @pl.when(i + b < n_my)
                def _():
                    g = i + b
                    pltpu.make_async_copy(a_hbm.at[idx1.at[b]],
                                          buf1.at[b], gsems[b]).wait()
                    pltpu.make_async_copy(b_hbm.at[idx2.at[b]],
                                          buf2.at[b], gsems[b]).wait()

                    @pl.when(g + 1 < n_my)
                    def _():
                        nb = base0 + (g + 1) * _CH
                        pltpu.sync_copy(src_hbm.at[pl.ds(nb, _CH)],
                                        idx1.at[nxt])
                        pltpu.sync_copy(dst_hbm.at[pl.ds(nb, _CH)],
                                        idx2.at[nxt])

                        @pl.when(g >= 1)
                        def _():
                            ob = base0 + (g - 1) * _CH
                            pltpu.make_async_copy(
                                buf1.at[nxt], hr_hbm.at[pl.ds(ob, _CH)],
                                wsems[nxt]).wait()
                            pltpu.make_async_copy(
                                buf2.at[nxt], hc_hbm.at[pl.ds(ob, _CH)],
                                wsems[nxt]).wait()

                        pltpu.async_copy(a_hbm.at[idx1.at[nxt]],
                                         buf1.at[nxt], gsems[nxt])
                        pltpu.async_copy(b_hbm.at[idx2.at[nxt]],
                                         buf2.at[nxt], gsems[nxt])

                    ob = base0 + g * _CH
                    pltpu.async_copy(buf1.at[b], hr_hbm.at[pl.ds(ob, _CH)],
                                     wsems[b])
                    pltpu.async_copy(buf2.at[b], hc_hbm.at[pl.ds(ob, _CH)],
                                     wsems[b])

        lb = base0 + (n_my - 1) * _CH
        pb = base0 + (n_my - 2) * _CH
        pltpu.make_async_copy(buf1.at[0], hr_hbm.at[pl.ds(pb, _CH)],
                              wsem0).wait()
        pltpu.make_async_copy(buf2.at[0], hc_hbm.at[pl.ds(pb, _CH)],
                              wsem0).wait()
        pltpu.make_async_copy(buf1.at[1], hr_hbm.at[pl.ds(lb, _CH)],
                              wsem1).wait()
        pltpu.make_async_copy(buf2.at[1], hc_hbm.at[pl.ds(lb, _CH)],
                              wsem1).wait()

    return k(a, b, src_pad, dst_pad)


def kernel(x, edge_index, edge_attr, W1, b1, W2, b2, We1, be1, We2, be2,
           We3, be3, We4, be4, Wp1, bp1, Wp2, bp2, Wp3, bp3):
    row = edge_index[0]
    col = edge_index[1]
    pad_idx = jnp.full((E_PAD - E,), N, jnp.int32)
    src_pad = jnp.concatenate([row, pad_idx])
    dst_pad = jnp.concatenate([col, pad_idx])

    x_pad = jnp.zeros((N_PAD, D_IN), jnp.float32).at[:N].set(x)
    ea_pad = jnp.zeros((E_PAD, D_EDGE), jnp.float32).at[:E].set(edge_attr)

    ones_blk = jnp.ones((_CH,), jnp.float32)
    zeros_vec = jnp.zeros((_RPT,), jnp.float32)
    zeros_blk = jnp.zeros((_RPT, H), jnp.float32)

    cnt = _sc_degree(dst_pad, ones_blk, zeros_vec)

    dis, hs1 = _tc_node1(cnt[:, :, None], x_pad, W1)
    agg1 = _sc_aggregate(hs1, src_pad, dst_pad, zeros_blk)
    hs2 = _tc_node2(agg1, hs1, dis, b1[None, :], W2)
    agg2 = _sc_aggregate(hs2, src_pad, dst_pad, zeros_blk)
    A, B = _tc_node3(agg2, hs2, dis, b2[None, :], Wp1[:H], Wp1[H:2 * H])

    hr, hc = _sc_gather_pair(A, B, src_pad, dst_pad)

    out = _tc_edge(ea_pad, hr, hc, We1, be1[None, :], We2, be2[None, :],
                   We3, be3[None, :], We4, be4[None, :],
                   Wp1[2 * H:], bp1[None, :], Wp2, bp2[None, :],
                   Wp3.T, bp3[None, :])
    return out[:E]


# speedup vs baseline: 6.4793x; 1.0104x over previous
import functools

import jax
import jax.numpy as jnp
from jax import lax
from jax.experimental import pallas as pl
from jax.experimental.pallas import tpu as pltpu
from jax.experimental.pallas import tpu_sc as plsc

N = 10000
E = 320000
D_IN = 128
D_EDGE = 16
H = 128

N_PAD = 10240
E_PAD = 323584
BE = 4096


def _node1_body(cnt_ref, x_ref, w1_ref, dis_ref, hs1_ref):
    cnt = cnt_ref[0] + cnt_ref[1]
    dis = jax.lax.rsqrt(cnt + 1.0)
    dis_ref[...] = dis
    hs1_ref[...] = dis * jnp.dot(x_ref[...], w1_ref[...],
                                 preferred_element_type=jnp.float32)


def _node2_body(agg_ref, hs1_ref, dis_ref, b1_ref, w2_ref, hs2_ref):
    agg = agg_ref[0] + agg_ref[1] + hs1_ref[...]
    dis = dis_ref[...]
    h1 = jax.nn.relu(dis * agg + b1_ref[...])
    hs2_ref[...] = dis * jnp.dot(h1, w2_ref[...],
                                 preferred_element_type=jnp.float32)


def _node3_body(agg_ref, hs2_ref, dis_ref, b2_ref, wpa_ref, wpb_ref,
                a_ref, b_ref):
    agg = agg_ref[0] + agg_ref[1] + hs2_ref[...]
    dis = dis_ref[...]
    h2 = jax.nn.relu(dis * agg + b2_ref[...])
    a_ref[...] = jnp.dot(h2, wpa_ref[...], preferred_element_type=jnp.float32)
    b_ref[...] = jnp.dot(h2, wpb_ref[...], preferred_element_type=jnp.float32)


H2 = H // 2


def _edge_body(ea_ref, hr_ref, hc_ref,
               we1_ref, be1_ref, we2_ref, be2_ref, we3_ref, be3_ref,
               we4_ref, be4_ref, wpc_ref, bp1_ref, wp2_ref, bp2_ref,
               wp3_ref, bp3_ref, out_ref):
    f32 = jnp.float32
    bf16 = jnp.bfloat16

    def mm(a, w_ref):
        return jnp.dot(a.astype(bf16), w_ref[...],
                       preferred_element_type=f32)

    e = jax.nn.relu(mm(ea_ref[...], we1_ref) + be1_ref[...])
    e = jax.nn.relu(mm(e, we2_ref) + be2_ref[...])
    e = jax.nn.relu(mm(e, we3_ref) + be3_ref[...])
    e = jax.nn.relu(mm(e, we4_ref) + be4_ref[...])
    p = jax.nn.relu(hr_ref[...] + hc_ref[...] + mm(e, wpc_ref)
                    + bp1_ref[...])
    p = jax.nn.relu(mm(p, wp2_ref) + bp2_ref[...])
    out_ref[...] = (jnp.sum(p * wp3_ref[...], axis=1, keepdims=True)
                    + bp3_ref[...])


def _full(shape):
    nd = len(shape)
    return pl.BlockSpec(shape, lambda *_: (0,) * nd)


def _tc_node1(cnt, x, W1):
    return pl.pallas_call(
        _node1_body,
        grid=(1,),
        in_specs=[_full((2, N_PAD, 1)), _full((N_PAD, D_IN)), _full((D_IN, H))],
        out_specs=[_full((N_PAD, 1)), _full((N_PAD, H))],
        out_shape=[jax.ShapeDtypeStruct((N_PAD, 1), jnp.float32),
                   jax.ShapeDtypeStruct((N_PAD, H), jnp.float32)],
    )(cnt, x, W1)


def _tc_node2(agg, hs1, dis, b1, W2):
    return pl.pallas_call(
        _node2_body,
        grid=(1,),
        in_specs=[_full((2, N_PAD, H)), _full((N_PAD, H)), _full((N_PAD, 1)),
                  _full((1, H)), _full((H, H))],
        out_specs=[_full((N_PAD, H))],
        out_shape=[jax.ShapeDtypeStruct((N_PAD, H), jnp.float32)],
    )(agg, hs1, dis, b1, W2)[0]


def _tc_node3(agg, hs2, dis, b2, Wpa, Wpb):
    return pl.pallas_call(
        _node3_body,
        grid=(1,),
        in_specs=[_full((2, N_PAD, H)), _full((N_PAD, H)), _full((N_PAD, 1)),
                  _full((1, H)), _full((H, H)), _full((H, H))],
        out_specs=[_full((N_PAD, H)), _full((N_PAD, H))],
        out_shape=[jax.ShapeDtypeStruct((N_PAD, H), jnp.float32),
                   jax.ShapeDtypeStruct((N_PAD, H), jnp.float32)],
    )(agg, hs2, dis, b2, Wpa, Wpb)


def _tc_edge(ea, hr, hc, We1, be1, We2, be2, We3, be3, We4, be4,
             Wpc, bp1, Wp2, bp2, wp3_row, bp3):
    nblk = E_PAD // BE
    eb = pl.BlockSpec((BE, D_EDGE), lambda i: (i, 0))
    hb = pl.BlockSpec((BE, H), lambda i: (i, 0))
    return pl.pallas_call(
        _edge_body,
        grid=(nblk,),
        in_specs=[eb, hb, hb,
                  _full((D_EDGE, H)), _full((1, H)),
                  _full((H, H)), _full((1, H)),
                  _full((H, H)), _full((1, H)),
                  _full((H, H)), _full((1, H)),
                  _full((H, H)), _full((1, H)),
                  _full((H, H)), _full((1, H)),
                  _full((1, H)), _full((1, 1))],
        out_specs=[pl.BlockSpec((BE, 1), lambda i: (i, 0))],
        out_shape=[jax.ShapeDtypeStruct((E_PAD, 1), jnp.float32)],
    )(ea, hr, hc, We1, be1, We2, be2, We3, be3, We4, be4,
      Wpc, bp1, Wp2, bp2, wp3_row, bp3)[0]


_CH = 128
_NW = 32
_RPT = N_PAD // 16

_NCH0 = 100
_NCH1 = 58
_MAXCH = max(_NCH0, _NCH1)


def _my_chunks(c, s):
    n_my = lax.select(c == 0, jnp.int32(_NCH0), jnp.int32(_NCH1))
    chunk0 = lax.select(c == 0, s * _NCH0, 16 * _NCH0 + s * _NCH1)
    return n_my, chunk0 * _CH


def _sc_mesh():
    return plsc.VectorSubcoreMesh(core_axis_name="c", subcore_axis_name="s")


def _sc_degree(dst_pad, ones_blk, zeros_vec):
    @functools.partial(
        pl.kernel, mesh=_sc_mesh(),
        out_type=jax.ShapeDtypeStruct((2, N_PAD), jnp.float32),
        scratch_types=[pltpu.VMEM((_CH,), jnp.int32),
                       pltpu.VMEM((_CH,), jnp.float32),
                       pltpu.VMEM_SHARED((N_PAD,), jnp.float32),
                       pltpu.SemaphoreType.DMA])
    def k(dst_hbm, ones_hbm, z_hbm, out_hbm, idx_d, ones_v, acc, sem):
        c = lax.axis_index("c")
        s = lax.axis_index("s")
        n_my, base0 = _my_chunks(c, s)
        pltpu.sync_copy(z_hbm, acc.at[pl.ds(s * _RPT, _RPT)])
        pltpu.sync_copy(ones_hbm, ones_v)
        plsc.subcore_barrier()

        @pl.loop(0, _MAXCH)
        def _(i):
            @pl.when(i < n_my)
            def _():
                base = base0 + i * _CH
                pltpu.sync_copy(dst_hbm.at[pl.ds(base, _CH)], idx_d)
                pltpu.sync_copy(ones_v, acc.at[idx_d], add=True)

        plsc.subcore_barrier()
        pltpu.sync_copy(acc.at[pl.ds(s * _RPT, _RPT)],
                        out_hbm.at[c, pl.ds(s * _RPT, _RPT)])

    return k(dst_pad, ones_blk, zeros_vec)


def _sc_aggregate(hs, src_pad, dst_pad, zeros_blk):
    @functools.partial(
        pl.kernel, mesh=_sc_mesh(),
        out_type=jax.ShapeDtypeStruct((2, N_PAD, H), jnp.float32),
        scratch_types=[pltpu.VMEM((2, _CH), jnp.int32),
                       pltpu.VMEM((2, _CH), jnp.int32),
                       pltpu.VMEM((2, _CH, H), jnp.float32),
                       pltpu.VMEM_SHARED((N_PAD, H), jnp.float32),
                       pltpu.SemaphoreType.DMA,
                       pltpu.SemaphoreType.DMA])
    def k(hs_hbm, src_hbm, dst_hbm, z_hbm, out_hbm, idx_s, idx_d, rows, acc,
          sem0, sem1):
        c = lax.axis_index("c")
        s = lax.axis_index("s")
        n_my, base0 = _my_chunks(c, s)
        sems = [sem0, sem1]
        pltpu.sync_copy(z_hbm, acc.at[pl.ds(s * _RPT, _RPT)])
        plsc.subcore_barrier()

        pltpu.sync_copy(src_hbm.at[pl.ds(base0, _CH)], idx_s.at[0])
        pltpu.sync_copy(dst_hbm.at[pl.ds(base0, _CH)], idx_d.at[0])
        pltpu.async_copy(hs_hbm.at[idx_s.at[0]], rows.at[0], sem0)

        @pl.loop(0, _MAXCH, step=2)
        def _(i):
            for b in range(2):
                nxt = 1 - b

                @pl.when(i + b < n_my)
                def _():
                    g = i + b
                    pltpu.make_async_copy(hs_hbm.at[idx_s.at[b]],
                                          rows.at[b], sems[b]).wait()

                    @pl.when(g + 1 < n_my)
                    def _():
                        nb = base0 + (g + 1) * _CH
                        pltpu.sync_copy(src_hbm.at[pl.ds(nb, _CH)],
                                        idx_s.at[nxt])
                        pltpu.sync_copy(dst_hbm.at[pl.ds(nb, _CH)],
                                        idx_d.at[nxt])
                        pltpu.async_copy(hs_hbm.at[idx_s.at[nxt]],
                                         rows.at[nxt], sems[nxt])

                    pltpu.sync_copy(rows.at[b], acc.at[idx_d.at[b]], add=True)

        plsc.subcore_barrier()
        pltpu.sync_copy(acc.at[pl.ds(s * _RPT, _RPT)],
                        out_hbm.at[c, pl.ds(s * _RPT, _RPT)])

    return k(hs, src_pad, dst_pad, zeros_blk)


def _sc_gather_pair(a, b, src_pad, dst_pad):
    @functools.partial(
        pl.kernel, mesh=_sc_mesh(),
        out_type=[jax.ShapeDtypeStruct((E_PAD, H), jnp.float32),
                  jax.ShapeDtypeStruct((E_PAD, H), jnp.float32)],
        scratch_types=[pltpu.VMEM((2, _CH), jnp.int32),
                       pltpu.VMEM((2, _CH), jnp.int32),
                       pltpu.VMEM((2, _CH, H), jnp.float32),
                       pltpu.VMEM((2, _CH, H), jnp.float32),
                       pltpu.SemaphoreType.DMA,
                       pltpu.SemaphoreType.DMA,
                       pltpu.SemaphoreType.DMA,
                       pltpu.SemaphoreType.DMA])
    def k(a_hbm, b_hbm, src_hbm, dst_hbm, hr_hbm, hc_hbm,
          idx1, idx2, buf1, buf2, gsem0, gsem1, wsem0, wsem1):
        c = lax.axis_index("c")
        s = lax.axis_index("s")
        n_my, base0 = _my_chunks(c, s)
        gsems = [gsem0, gsem1]
        wsems = [wsem0, wsem1]

        pltpu.sync_copy(src_hbm.at[pl.ds(base0, _CH)], idx1.at[0])
        pltpu.sync_copy(dst_hbm.at[pl.ds(base0, _CH)], idx2.at[0])
        pltpu.async_copy(a_hbm.at[idx1.at[0]], buf1.at[0], gsem0)
        pltpu.async_copy(b_hbm.at[idx2.at[0]], buf2.at[0], gsem0)

        @pl.loop(0, _MAXCH, step=2)
        def _(i):
            for b in range(2):
                nxt = 1 - b

                @pl.when(i + b < n_my)
                def _():
                    g = i + b
                    pltpu.make_async_copy(a_hbm.at[idx1.at[b]],
                                          buf1.at[b], gsems[b]).wait()
                    pltpu.make_async_copy(b_hbm.at[idx2.at[b]],
                                          buf2.at[b], gsems[b]).wait()

                    @pl.when(g + 1 < n_my)
                    def _():
                        nb = base0 + (g + 1) * _CH
                        pltpu.sync_copy(src_hbm.at[pl.ds(nb, _CH)],
                                        idx1.at[nxt])
                        pltpu.sync_copy(dst_hbm.at[pl.ds(nb, _CH)],
                                        idx2.at[nxt])

                        @pl.when(g >= 1)
                        def _():
                            ob = base0 + (g - 1) * _CH
                            pltpu.make_async_copy(
                                buf1.at[nxt], hr_hbm.at[pl.ds(ob, _CH)],
                                wsems[nxt]).wait()
                            pltpu.make_async_copy(
                                buf2.at[nxt], hc_hbm.at[pl.ds(ob, _CH)],
                                wsems[nxt]).wait()

                        pltpu.async_copy(a_hbm.at[idx1.at[nxt]],
                                         buf1.at[nxt], gsems[nxt])
                        pltpu.async_copy(b_hbm.at[idx2.at[nxt]],
                                         buf2.at[nxt], gsems[nxt])

                    ob = base0 + g * _CH
                    pltpu.async_copy(buf1.at[b], hr_hbm.at[pl.ds(ob, _CH)],
                                     wsems[b])
                    pltpu.async_copy(buf2.at[b], hc_hbm.at[pl.ds(ob, _CH)],
                                     wsems[b])

        lb = base0 + (n_my - 1) * _CH
        pb = base0 + (n_my - 2) * _CH
        pltpu.make_async_copy(buf1.at[0], hr_hbm.at[pl.ds(pb, _CH)],
                              wsem0).wait()
        pltpu.make_async_copy(buf2.at[0], hc_hbm.at[pl.ds(pb, _CH)],
                              wsem0).wait()
        pltpu.make_async_copy(buf1.at[1], hr_hbm.at[pl.ds(lb, _CH)],
                              wsem1).wait()
        pltpu.make_async_copy(buf2.at[1], hc_hbm.at[pl.ds(lb, _CH)],
                              wsem1).wait()

    return k(a, b, src_pad, dst_pad)


def kernel(x, edge_index, edge_attr, W1, b1, W2, b2, We1, be1, We2, be2,
           We3, be3, We4, be4, Wp1, bp1, Wp2, bp2, Wp3, bp3):
    row = edge_index[0]
    col = edge_index[1]
    pad_idx = jnp.full((E_PAD - E,), N, jnp.int32)
    src_pad = jnp.concatenate([row, pad_idx])
    dst_pad = jnp.concatenate([col, pad_idx])

    x_pad = jnp.zeros((N_PAD, D_IN), jnp.float32).at[:N].set(x)
    ea_pad = jnp.zeros((E_PAD, D_EDGE), jnp.float32).at[:E].set(edge_attr)

    ones_blk = jnp.ones((_CH,), jnp.float32)
    zeros_vec = jnp.zeros((_RPT,), jnp.float32)
    zeros_blk = jnp.zeros((_RPT, H), jnp.float32)

    cnt = _sc_degree(dst_pad, ones_blk, zeros_vec)

    dis, hs1 = _tc_node1(cnt[:, :, None], x_pad, W1)
    agg1 = _sc_aggregate(hs1, src_pad, dst_pad, zeros_blk)
    hs2 = _tc_node2(agg1, hs1, dis, b1[None, :], W2)
    agg2 = _sc_aggregate(hs2, src_pad, dst_pad, zeros_blk)
    A, B = _tc_node3(agg2, hs2, dis, b2[None, :], Wp1[:H], Wp1[H:2 * H])

    hr, hc = _sc_gather_pair(A, B, src_pad, dst_pad)

    bf = jnp.bfloat16
    out = _tc_edge(ea_pad, hr, hc, We1.astype(bf), be1[None, :],
                   We2.astype(bf), be2[None, :], We3.astype(bf),
                   be3[None, :], We4.astype(bf), be4[None, :],
                   Wp1[2 * H:].astype(bf), bp1[None, :], Wp2.astype(bf),
                   bp2[None, :], Wp3.T, bp3[None, :])
    return out[:E]
